# baseline (speedup 1.0000x reference)
import jax
import jax.numpy as jnp
from jax import lax
from jax.experimental import pallas as pl
from jax.experimental.pallas import tpu as pltpu

B, S, D = 16, 4096, 1024
BS = 2048
NEG_INF = float("-inf")


def _firstpos_body(sot_ref, fp_ref):
    is_sot = sot_ref[...] != 0
    iota = lax.broadcasted_iota(jnp.int32, (B, S), 1)
    fp = jnp.min(jnp.where(is_sot, iota, S), axis=1, keepdims=True)
    fp_ref[...] = jnp.where(fp == S, 0, fp)


def _gather_body(fp_ref, seq_ref, out_ref):
    b = pl.program_id(0)
    r = fp_ref[b] % 8
    out_ref[...] = seq_ref[:, pl.ds(r, 1), :]


def _stream_body(seq_ref, spk_ref, dot_ref, nsq_ref):
    x = seq_ref[0]
    spk = spk_ref[0]
    dot_ref[...] = jnp.sum(x * spk, axis=1)[None, None, :]
    nsq_ref[...] = jnp.sum(x * x, axis=1)[None, None, :]


def _cumsum_lastdim(x):
    k = 1
    while k < S:
        shifted = jnp.concatenate(
            [jnp.zeros((B, k), x.dtype), x[:, : S - k]], axis=1)
        x = x + shifted
        k *= 2
    return x


def _finalize_body(dot_ref, nsq_ref, sot_ref, labels_ref, spk_ref,
                   loss_ref, pred_ref):
    dot = dot_ref[...]
    nsq = nsq_ref[...]
    is_sot = sot_ref[...] != 0
    labels = labels_ref[...]
    spk = spk_ref[...]

    cs = _cumsum_lastdim(is_sot.astype(jnp.int32))
    spk_nsq = jnp.sum(spk * spk, axis=2)
    denom = jnp.maximum(jnp.sqrt(nsq) * jnp.sqrt(spk_nsq), 1e-6)
    sim = dot / denom
    remain = is_sot & (cs >= 2)
    simm = jnp.where(remain, sim, NEG_INF)

    m = jnp.max(simm, axis=1, keepdims=True)
    lse = m + jnp.log(jnp.sum(jnp.exp(simm - m), axis=1, keepdims=True))

    lmask = is_sot & (cs == labels + 2)
    has_l = jnp.any(lmask, axis=1, keepdims=True)
    val_l = jnp.sum(jnp.where(lmask, simm, 0.0), axis=1, keepdims=True)
    logp = jnp.where(has_l, val_l, simm[:, 0:1]) - lse
    loss_ref[...] = jnp.mean(-logp)[None, None]

    iota = lax.broadcasted_iota(jnp.int32, (B, S), 1)
    ppos = jnp.min(jnp.where(simm == m, iota, S), axis=1, keepdims=True)
    ppos = jnp.where(ppos == S, 0, ppos)
    pcs = jnp.sum(jnp.where(iota == ppos, cs, 0), axis=1, keepdims=True)
    pred_ref[...] = pcs - 2


def kernel(sequence_output, sot_positions, labels):
    sot_positions = sot_positions.astype(jnp.int32)

    first_pos = pl.pallas_call(
        _firstpos_body,
        out_shape=jax.ShapeDtypeStruct((B, 1), jnp.int32),
    )(sot_positions)

    seq_rows8 = sequence_output.reshape(B * S // 8, 8, D)
    speakers = pl.pallas_call(
        _gather_body,
        grid_spec=pltpu.PrefetchScalarGridSpec(
            num_scalar_prefetch=1,
            grid=(B,),
            in_specs=[pl.BlockSpec(
                (1, 8, D), lambda b, fp: ((b * S + fp[b]) // 8, 0, 0))],
            out_specs=pl.BlockSpec((1, 1, D), lambda b, fp: (b, 0, 0)),
        ),
        out_shape=jax.ShapeDtypeStruct((B, 1, D), jnp.float32),
    )(first_pos.reshape(B), seq_rows8)

    nsb = S // BS
    dot, nsq = pl.pallas_call(
        _stream_body,
        grid=(B, nsb),
        in_specs=[
            pl.BlockSpec((1, BS, D), lambda b, s: (b, s, 0)),
            pl.BlockSpec((1, 1, D), lambda b, s: (b, 0, 0)),
        ],
        out_specs=[
            pl.BlockSpec((1, 1, BS), lambda b, s: (b * nsb + s, 0, 0)),
            pl.BlockSpec((1, 1, BS), lambda b, s: (b * nsb + s, 0, 0)),
        ],
        out_shape=[
            jax.ShapeDtypeStruct((B * nsb, 1, BS), jnp.float32),
            jax.ShapeDtypeStruct((B * nsb, 1, BS), jnp.float32),
        ],
    )(sequence_output, speakers)
    dot = dot.reshape(B, S)
    nsq = nsq.reshape(B, S)

    loss, pred = pl.pallas_call(
        _finalize_body,
        out_shape=[
            jax.ShapeDtypeStruct((1, 1), jnp.float32),
            jax.ShapeDtypeStruct((B, 1), jnp.int32),
        ],
    )(dot, nsq, sot_positions, labels.astype(jnp.int32).reshape(B, 1),
      speakers)

    return (loss[0, 0], pred.reshape(B), labels.astype(jnp.int32))


# speedup vs baseline: 1.0945x; 1.0945x over previous
import jax
import jax.numpy as jnp
from jax import lax
from jax.experimental import pallas as pl
from jax.experimental.pallas import tpu as pltpu

B, S, D = 16, 4096, 1024
BS = 4096
NEG_INF = float("-inf")


def _firstpos_body(sot_ref, fp_ref):
    is_sot = sot_ref[...] != 0
    iota = lax.broadcasted_iota(jnp.int32, (B, S), 1)
    fp = jnp.min(jnp.where(is_sot, iota, S), axis=1, keepdims=True)
    fp_ref[...] = jnp.where(fp == S, 0, fp)


def _gather_body(fp_ref, seq_ref, out_ref):
    b = pl.program_id(0)
    r = fp_ref[b] % 8
    out_ref[...] = seq_ref[:, pl.ds(r, 1), :]


def _stream_body(seq_ref, spk_ref, dot_ref, nsq_ref):
    x = seq_ref[0]
    spk = spk_ref[0]
    dot_ref[...] = jnp.sum(x * spk, axis=1)[None, None, :]
    nsq_ref[...] = jnp.sum(x * x, axis=1)[None, None, :]


def _cumsum_lastdim(x):
    k = 1
    while k < S:
        shifted = jnp.concatenate(
            [jnp.zeros((B, k), x.dtype), x[:, : S - k]], axis=1)
        x = x + shifted
        k *= 2
    return x


def _finalize_body(dot_ref, nsq_ref, sot_ref, labels_ref, spk_ref,
                   loss_ref, pred_ref):
    dot = dot_ref[...]
    nsq = nsq_ref[...]
    is_sot = sot_ref[...] != 0
    labels = labels_ref[...]
    spk = spk_ref[...]

    cs = _cumsum_lastdim(is_sot.astype(jnp.int32))
    spk_nsq = jnp.sum(spk * spk, axis=2)
    denom = jnp.maximum(jnp.sqrt(nsq) * jnp.sqrt(spk_nsq), 1e-6)
    sim = dot / denom
    remain = is_sot & (cs >= 2)
    simm = jnp.where(remain, sim, NEG_INF)

    m = jnp.max(simm, axis=1, keepdims=True)
    lse = m + jnp.log(jnp.sum(jnp.exp(simm - m), axis=1, keepdims=True))

    lmask = is_sot & (cs == labels + 2)
    has_l = jnp.any(lmask, axis=1, keepdims=True)
    val_l = jnp.sum(jnp.where(lmask, simm, 0.0), axis=1, keepdims=True)
    logp = jnp.where(has_l, val_l, simm[:, 0:1]) - lse
    loss_ref[...] = jnp.mean(-logp)[None, None]

    iota = lax.broadcasted_iota(jnp.int32, (B, S), 1)
    ppos = jnp.min(jnp.where(simm == m, iota, S), axis=1, keepdims=True)
    ppos = jnp.where(ppos == S, 0, ppos)
    pcs = jnp.sum(jnp.where(iota == ppos, cs, 0), axis=1, keepdims=True)
    pred_ref[...] = pcs - 2


def kernel(sequence_output, sot_positions, labels):
    sot_positions = sot_positions.astype(jnp.int32)

    first_pos = pl.pallas_call(
        _firstpos_body,
        out_shape=jax.ShapeDtypeStruct((B, 1), jnp.int32),
    )(sot_positions)

    seq_rows8 = sequence_output.reshape(B * S // 8, 8, D)
    speakers = pl.pallas_call(
        _gather_body,
        grid_spec=pltpu.PrefetchScalarGridSpec(
            num_scalar_prefetch=1,
            grid=(B,),
            in_specs=[pl.BlockSpec(
                (1, 8, D), lambda b, fp: ((b * S + fp[b]) // 8, 0, 0))],
            out_specs=pl.BlockSpec((1, 1, D), lambda b, fp: (b, 0, 0)),
        ),
        out_shape=jax.ShapeDtypeStruct((B, 1, D), jnp.float32),
    )(first_pos.reshape(B), seq_rows8)

    nsb = S // BS
    dot, nsq = pl.pallas_call(
        _stream_body,
        grid=(B, nsb),
        in_specs=[
            pl.BlockSpec((1, BS, D), lambda b, s: (b, s, 0)),
            pl.BlockSpec((1, 1, D), lambda b, s: (b, 0, 0)),
        ],
        out_specs=[
            pl.BlockSpec((1, 1, BS), lambda b, s: (b * nsb + s, 0, 0)),
            pl.BlockSpec((1, 1, BS), lambda b, s: (b * nsb + s, 0, 0)),
        ],
        out_shape=[
            jax.ShapeDtypeStruct((B * nsb, 1, BS), jnp.float32),
            jax.ShapeDtypeStruct((B * nsb, 1, BS), jnp.float32),
        ],
    )(sequence_output, speakers)
    dot = dot.reshape(B, S)
    nsq = nsq.reshape(B, S)

    loss, pred = pl.pallas_call(
        _finalize_body,
        out_shape=[
            jax.ShapeDtypeStruct((1, 1), jnp.float32),
            jax.ShapeDtypeStruct((B, 1), jnp.int32),
        ],
    )(dot, nsq, sot_positions, labels.astype(jnp.int32).reshape(B, 1),
      speakers)

    return (loss[0, 0], pred.reshape(B), labels.astype(jnp.int32))


# speedup vs baseline: 1.2634x; 1.1543x over previous
import jax
import jax.numpy as jnp
from jax import lax
from jax.experimental import pallas as pl
from jax.experimental.pallas import tpu as pltpu

B, S, D = 16, 4096, 1024
BS = 128
NEG_INF = float("-inf")


def _firstpos_body(sot_ref, fp_ref):
    is_sot = sot_ref[...] != 0
    iota = lax.broadcasted_iota(jnp.int32, (B, S), 1)
    fp = jnp.min(jnp.where(is_sot, iota, S), axis=1, keepdims=True)
    fp_ref[...] = jnp.where(fp == S, 0, fp)


def _gather_body(fp_ref, seq_ref, out_ref):
    b = pl.program_id(0)
    r = fp_ref[b] % 8
    out_ref[...] = seq_ref[:, pl.ds(r, 1), :]


def _stream_body(seq_ref, spk_ref, dot_ref, nsq_ref):
    x = seq_ref[...]
    spk = spk_ref[...]
    dot_ref[...] = jnp.sum(x * spk, axis=2)
    nsq_ref[...] = jnp.sum(x * x, axis=2)


def _cumsum_lastdim(x):
    k = 1
    while k < S:
        shifted = jnp.concatenate(
            [jnp.zeros((B, k), x.dtype), x[:, : S - k]], axis=1)
        x = x + shifted
        k *= 2
    return x


def _finalize_body(dot_ref, nsq_ref, sot_ref, labels_ref, spk_ref,
                   loss_ref, pred_ref):
    dot = dot_ref[...]
    nsq = nsq_ref[...]
    is_sot = sot_ref[...] != 0
    labels = labels_ref[...]
    spk = spk_ref[...]

    cs = _cumsum_lastdim(is_sot.astype(jnp.int32))
    spk_nsq = jnp.sum(spk * spk, axis=2)
    denom = jnp.maximum(jnp.sqrt(nsq) * jnp.sqrt(spk_nsq), 1e-6)
    sim = dot / denom
    remain = is_sot & (cs >= 2)
    simm = jnp.where(remain, sim, NEG_INF)

    m = jnp.max(simm, axis=1, keepdims=True)
    lse = m + jnp.log(jnp.sum(jnp.exp(simm - m), axis=1, keepdims=True))

    lmask = is_sot & (cs == labels + 2)
    has_l = jnp.any(lmask, axis=1, keepdims=True)
    val_l = jnp.sum(jnp.where(lmask, simm, 0.0), axis=1, keepdims=True)
    logp = jnp.where(has_l, val_l, simm[:, 0:1]) - lse
    loss_ref[...] = jnp.mean(-logp)[None, None]

    iota = lax.broadcasted_iota(jnp.int32, (B, S), 1)
    ppos = jnp.min(jnp.where(simm == m, iota, S), axis=1, keepdims=True)
    ppos = jnp.where(ppos == S, 0, ppos)
    pcs = jnp.sum(jnp.where(iota == ppos, cs, 0), axis=1, keepdims=True)
    pred_ref[...] = pcs - 2


def kernel(sequence_output, sot_positions, labels):
    sot_positions = sot_positions.astype(jnp.int32)

    first_pos = pl.pallas_call(
        _firstpos_body,
        out_shape=jax.ShapeDtypeStruct((B, 1), jnp.int32),
    )(sot_positions)

    seq_rows8 = sequence_output.reshape(B * S // 8, 8, D)
    speakers = pl.pallas_call(
        _gather_body,
        grid_spec=pltpu.PrefetchScalarGridSpec(
            num_scalar_prefetch=1,
            grid=(B,),
            in_specs=[pl.BlockSpec(
                (1, 8, D), lambda b, fp: ((b * S + fp[b]) // 8, 0, 0))],
            out_specs=pl.BlockSpec((1, 1, D), lambda b, fp: (b, 0, 0)),
        ),
        out_shape=jax.ShapeDtypeStruct((B, 1, D), jnp.float32),
    )(first_pos.reshape(B), seq_rows8)

    dot, nsq = pl.pallas_call(
        _stream_body,
        grid=(S // BS,),
        in_specs=[
            pl.BlockSpec((B, BS, D), lambda s: (0, s, 0)),
            pl.BlockSpec((B, 1, D), lambda s: (0, 0, 0)),
        ],
        out_specs=[
            pl.BlockSpec((B, BS), lambda s: (0, s)),
            pl.BlockSpec((B, BS), lambda s: (0, s)),
        ],
        out_shape=[
            jax.ShapeDtypeStruct((B, S), jnp.float32),
            jax.ShapeDtypeStruct((B, S), jnp.float32),
        ],
    )(sequence_output, speakers)

    loss, pred = pl.pallas_call(
        _finalize_body,
        out_shape=[
            jax.ShapeDtypeStruct((1, 1), jnp.float32),
            jax.ShapeDtypeStruct((B, 1), jnp.int32),
        ],
    )(dot, nsq, sot_positions, labels.astype(jnp.int32).reshape(B, 1),
      speakers)

    return (loss[0, 0], pred.reshape(B), labels.astype(jnp.int32))


# speedup vs baseline: 1.2810x; 1.0140x over previous
import jax
import jax.numpy as jnp
from jax import lax
from jax.experimental import pallas as pl
from jax.experimental.pallas import tpu as pltpu

B, S, D = 16, 4096, 1024
BS = 128
NSB = S // BS
NEG_INF = float("-inf")


def _firstpos_body(sot_ref, fp_ref):
    is_sot = sot_ref[...] != 0
    iota = lax.broadcasted_iota(jnp.int32, (B, S), 1)
    fp = jnp.min(jnp.where(is_sot, iota, S), axis=1, keepdims=True)
    fp_ref[...] = jnp.where(fp == S, 0, fp)


def _gather_body(fp_ref, seq_ref, out_ref):
    b = pl.program_id(0)
    r = fp_ref[b] % 8
    out_ref[...] = seq_ref[:, pl.ds(r, 1), :]


def _cumsum_lastdim(x):
    k = 1
    while k < S:
        shifted = jnp.concatenate(
            [jnp.zeros((B, k), x.dtype), x[:, : S - k]], axis=1)
        x = x + shifted
        k *= 2
    return x


def _stream_body(seq_ref, spk_ref, sot_ref, labels_ref,
                 loss_ref, pred_ref, dot_acc, nsq_acc):
    s = pl.program_id(0)
    x = seq_ref[...]
    spk = spk_ref[...]
    dot_acc[:, pl.ds(s * BS, BS)] = jnp.sum(x * spk, axis=2)
    nsq_acc[:, pl.ds(s * BS, BS)] = jnp.sum(x * x, axis=2)

    @pl.when(s == NSB - 1)
    def _finalize():
        dot = dot_acc[...]
        nsq = nsq_acc[...]
        is_sot = sot_ref[...] != 0
        labels = labels_ref[...]

        cs = _cumsum_lastdim(is_sot.astype(jnp.int32))
        spk_nsq = jnp.sum(spk * spk, axis=2)
        denom = jnp.maximum(jnp.sqrt(nsq) * jnp.sqrt(spk_nsq), 1e-6)
        sim = dot / denom
        remain = is_sot & (cs >= 2)
        simm = jnp.where(remain, sim, NEG_INF)

        m = jnp.max(simm, axis=1, keepdims=True)
        lse = m + jnp.log(jnp.sum(jnp.exp(simm - m), axis=1, keepdims=True))

        lmask = is_sot & (cs == labels + 2)
        has_l = jnp.any(lmask, axis=1, keepdims=True)
        val_l = jnp.sum(jnp.where(lmask, simm, 0.0), axis=1, keepdims=True)
        logp = jnp.where(has_l, val_l, simm[:, 0:1]) - lse
        loss_ref[...] = jnp.mean(-logp)[None, None]

        iota = lax.broadcasted_iota(jnp.int32, (B, S), 1)
        ppos = jnp.min(jnp.where(simm == m, iota, S), axis=1, keepdims=True)
        ppos = jnp.where(ppos == S, 0, ppos)
        pcs = jnp.sum(jnp.where(iota == ppos, cs, 0), axis=1, keepdims=True)
        pred_ref[...] = pcs - 2


def kernel(sequence_output, sot_positions, labels):
    sot_positions = sot_positions.astype(jnp.int32)

    first_pos = pl.pallas_call(
        _firstpos_body,
        out_shape=jax.ShapeDtypeStruct((B, 1), jnp.int32),
    )(sot_positions)

    seq_rows8 = sequence_output.reshape(B * S // 8, 8, D)
    speakers = pl.pallas_call(
        _gather_body,
        grid_spec=pltpu.PrefetchScalarGridSpec(
            num_scalar_prefetch=1,
            grid=(B,),
            in_specs=[pl.BlockSpec(
                (1, 8, D), lambda b, fp: ((b * S + fp[b]) // 8, 0, 0))],
            out_specs=pl.BlockSpec((1, 1, D), lambda b, fp: (b, 0, 0)),
        ),
        out_shape=jax.ShapeDtypeStruct((B, 1, D), jnp.float32),
    )(first_pos.reshape(B), seq_rows8)

    loss, pred = pl.pallas_call(
        _stream_body,
        grid=(NSB,),
        in_specs=[
            pl.BlockSpec((B, BS, D), lambda s: (0, s, 0)),
            pl.BlockSpec((B, 1, D), lambda s: (0, 0, 0)),
            pl.BlockSpec((B, S), lambda s: (0, 0)),
            pl.BlockSpec((B, 1), lambda s: (0, 0)),
        ],
        out_specs=[
            pl.BlockSpec((1, 1), lambda s: (0, 0)),
            pl.BlockSpec((B, 1), lambda s: (0, 0)),
        ],
        out_shape=[
            jax.ShapeDtypeStruct((1, 1), jnp.float32),
            jax.ShapeDtypeStruct((B, 1), jnp.int32),
        ],
        scratch_shapes=[
            pltpu.VMEM((B, S), jnp.float32),
            pltpu.VMEM((B, S), jnp.float32),
        ],
    )(sequence_output, speakers, sot_positions,
      labels.astype(jnp.int32).reshape(B, 1))

    return (loss[0, 0], pred.reshape(B), labels.astype(jnp.int32))
